# baseline (speedup 1.0000x reference)
import functools

import jax
import jax.numpy as jnp
from jax import lax
from jax.experimental import pallas as pl
from jax.experimental.pallas import tpu as pltpu
from jax.experimental.pallas import tpu_sc as plsc

_CHUNK = 128
_LANES = 16


@functools.lru_cache(maxsize=None)
def _build_emb_kernel(B, V, D):
    info = plsc.get_sparse_core_info()
    num_workers = info.num_cores * info.num_subcores
    b_per_w = B // num_workers
    n_chunks = b_per_w // _CHUNK
    n_groups = b_per_w // _LANES

    mesh = plsc.VectorSubcoreMesh(core_axis_name="c", subcore_axis_name="s")

    @functools.partial(
        pl.kernel,
        mesh=mesh,
        out_type=jax.ShapeDtypeStruct((D, B), jnp.float32),
        scratch_types=[
            pltpu.VMEM((b_per_w + _LANES,), jnp.int32),
            pltpu.VMEM((b_per_w,), jnp.int32),
            pltpu.VMEM((b_per_w, 2 * D), jnp.float32),
            pltpu.VMEM((D, b_per_w + 1), jnp.float32),
            pltpu.SemaphoreType.DMA,
        ],
        compiler_params=pltpu.CompilerParams(
            use_tc_tiling_on_sc=True, needs_layout_passes=False
        ),
    )
    def emb(idx_hbm, table_hbm, out_hbm, idx_v, idx2_v, rows_v, rows_t, sem):
        wid = lax.axis_index("s") * info.num_cores + lax.axis_index("c")
        base = wid * b_per_w
        pltpu.sync_copy(idx_hbm.at[pl.ds(base, b_per_w)], idx_v.at[pl.ds(0, b_per_w)])
        for g in range(n_groups):
            sl = pl.ds(g * _LANES, _LANES)
            idx2_v[sl] = idx_v[sl] >> 1
        copies = [
            pltpu.async_copy(
                table_hbm.at[idx2_v.at[pl.ds(j * _CHUNK, _CHUNK)]],
                rows_v.at[pl.ds(j * _CHUNK, _CHUNK)],
                sem,
            )
            for j in range(n_chunks)
        ]
        for c in copies:
            c.wait()

        lane = lax.iota(jnp.int32, _LANES)
        d_vecs = [d0 * _LANES + lane for d0 in range(D // _LANES)]

        @plsc.parallel_loop(0, b_per_w)
        def transpose_row(b):
            par64 = (idx_v[pl.ds(b, _LANES)][0] & 1) * D
            b_vec = jnp.full((_LANES,), b, jnp.int32)
            for d0 in range(D // _LANES):
                vals = rows_v[b, pl.ds(par64 + d0 * _LANES, _LANES)]
                plsc.store_scatter(rows_t, [d_vecs[d0], b_vec], vals)

        pltpu.sync_copy(
            rows_t.at[:, pl.ds(0, b_per_w)],
            out_hbm.at[:, pl.ds(base, b_per_w)],
        )

    return emb


def kernel(disaster_type_idx, embedding_weight):
    (B,) = disaster_type_idx.shape
    V, D = embedding_weight.shape
    emb = _build_emb_kernel(B, V, D)
    table2 = embedding_weight.reshape(V // 2, 2 * D)
    out_t = emb(disaster_type_idx.astype(jnp.int32), table2)
    return out_t.T


# speedup vs baseline: 1.0976x; 1.0976x over previous
import functools

import jax
import jax.numpy as jnp
from jax import lax
from jax.experimental import pallas as pl
from jax.experimental.pallas import tpu as pltpu
from jax.experimental.pallas import tpu_sc as plsc

_CHUNK = 128
_LANES = 16


@functools.lru_cache(maxsize=None)
def _build_emb_kernel(B, V, D):
    info = plsc.get_sparse_core_info()
    num_workers = info.num_cores * info.num_subcores
    b_per_w = B // num_workers
    n_chunks = b_per_w // _CHUNK
    n_groups = b_per_w // _LANES

    mesh = plsc.VectorSubcoreMesh(core_axis_name="c", subcore_axis_name="s")

    @functools.partial(
        pl.kernel,
        mesh=mesh,
        out_type=jax.ShapeDtypeStruct((D, B), jnp.float32),
        scratch_types=[
            pltpu.VMEM((b_per_w,), jnp.int32),
            pltpu.VMEM((b_per_w, D), jnp.float32),
            pltpu.VMEM((D, b_per_w + 1), jnp.float32),
            pltpu.SemaphoreType.DMA,
        ],
        compiler_params=pltpu.CompilerParams(
            use_tc_tiling_on_sc=False, needs_layout_passes=False
        ),
    )
    def emb(idx_hbm, table_hbm, out_hbm, idx_v, rows_v, rows_t, sem):
        wid = lax.axis_index("s") * info.num_cores + lax.axis_index("c")
        base = wid * b_per_w
        pltpu.sync_copy(idx_hbm.at[pl.ds(base, b_per_w)], idx_v)
        copies = [
            pltpu.async_copy(
                table_hbm.at[idx_v.at[pl.ds(j * _CHUNK, _CHUNK)]],
                rows_v.at[pl.ds(j * _CHUNK, _CHUNK)],
                sem,
            )
            for j in range(n_chunks)
        ]
        for c in copies:
            c.wait()

        lane = lax.iota(jnp.int32, _LANES)
        d_vecs = [d0 * _LANES + lane for d0 in range(D // _LANES)]

        @plsc.parallel_loop(0, b_per_w)
        def transpose_row(b):
            b_vec = jnp.full((_LANES,), b, jnp.int32)
            for d0 in range(D // _LANES):
                vals = rows_v[b, pl.ds(d0 * _LANES, _LANES)]
                plsc.store_scatter(rows_t, [d_vecs[d0], b_vec], vals)

        pltpu.sync_copy(
            rows_t.at[:, pl.ds(0, b_per_w)],
            out_hbm.at[:, pl.ds(base, b_per_w)],
        )

    return emb


def kernel(disaster_type_idx, embedding_weight):
    (B,) = disaster_type_idx.shape
    V, D = embedding_weight.shape
    emb = _build_emb_kernel(B, V, D)
    out_t = emb(disaster_type_idx.astype(jnp.int32), embedding_weight)
    return out_t.T


# speedup vs baseline: 1.0976x; 1.0000x over previous
import functools

import jax
import jax.numpy as jnp
from jax import lax
from jax.experimental import pallas as pl
from jax.experimental.pallas import tpu as pltpu
from jax.experimental.pallas import tpu_sc as plsc

_CHUNK = 128
_LANES = 16


@functools.lru_cache(maxsize=None)
def _build_emb_kernel(B, V, D):
    info = plsc.get_sparse_core_info()
    num_workers = info.num_cores * info.num_subcores
    b_per_w = B // num_workers
    n_chunks = b_per_w // _CHUNK
    n_groups = b_per_w // _LANES

    mesh = plsc.VectorSubcoreMesh(core_axis_name="c", subcore_axis_name="s")

    @functools.partial(
        pl.kernel,
        mesh=mesh,
        out_type=jax.ShapeDtypeStruct((D, B), jnp.float32),
        scratch_types=[
            pltpu.VMEM((b_per_w,), jnp.int32),
            pltpu.VMEM((b_per_w, D), jnp.float32),
            pltpu.VMEM((D, b_per_w + 1), jnp.float32),
            pltpu.SemaphoreType.DMA,
            pltpu.SemaphoreType.DMA,
        ],
        compiler_params=pltpu.CompilerParams(
            use_tc_tiling_on_sc=False, needs_layout_passes=False
        ),
    )
    def emb(idx_hbm, table_hbm, out_hbm, idx_v, rows_v, rows_t, sem, out_sem):
        wid = lax.axis_index("s") * info.num_cores + lax.axis_index("c")
        base = wid * b_per_w
        pltpu.sync_copy(idx_hbm.at[pl.ds(base, b_per_w)], idx_v)

        def start_gather(j):
            return pltpu.async_copy(
                table_hbm.at[idx_v.at[pl.ds(j * _CHUNK, _CHUNK)]],
                rows_v.at[pl.ds(j * _CHUNK, _CHUNK)],
                sem,
            )

        lane = lax.iota(jnp.int32, _LANES)
        d_vecs = [d0 * _LANES + lane for d0 in range(D // _LANES)]

        pending = start_gather(0)
        out_copies = []
        for j in range(n_chunks):
            nxt = start_gather(j + 1) if j + 1 < n_chunks else None
            pending.wait()

            @plsc.parallel_loop(j * _CHUNK, (j + 1) * _CHUNK)
            def transpose_row(b):
                b_vec = jnp.full((_LANES,), b, jnp.int32)
                for d0 in range(D // _LANES):
                    vals = rows_v[b, pl.ds(d0 * _LANES, _LANES)]
                    plsc.store_scatter(rows_t, [d_vecs[d0], b_vec], vals)

            out_copies.append(
                pltpu.async_copy(
                    rows_t.at[:, pl.ds(j * _CHUNK, _CHUNK)],
                    out_hbm.at[:, pl.ds(base + j * _CHUNK, _CHUNK)],
                    out_sem,
                )
            )
            pending = nxt
        for c in out_copies:
            c.wait()

    return emb


def kernel(disaster_type_idx, embedding_weight):
    (B,) = disaster_type_idx.shape
    V, D = embedding_weight.shape
    emb = _build_emb_kernel(B, V, D)
    out_t = emb(disaster_type_idx.astype(jnp.int32), embedding_weight)
    return out_t.T
